# baseline (speedup 1.0000x reference)
import jax
import jax.numpy as jnp
from jax.experimental import pallas as pl
from jax.experimental.pallas import tpu as pltpu

_EPS = 1e-5
_N_CHUNKS = 10


def _bn_relu_chain_kernel(x_ref, g_ref, b_ref, o_ref,
                          h_ref, s1_ref, s2_ref, sc_ref, sh_ref):
    p = pl.program_id(0)
    c = pl.program_id(1)
    n_rows = jnp.float32(h_ref.shape[0])
    chunk = x_ref.shape[0]
    n_phases = pl.num_programs(0)

    @pl.when(c == 0)
    def _start_phase():
        @pl.when(p > 0)
        def _finalize_stats():
            mean = s1_ref[...] / n_rows
            var = s2_ref[...] / n_rows - mean * mean
            scale = jax.lax.rsqrt(var + _EPS) * g_ref[pl.ds(p - 1, 1), :]
            sc_ref[...] = scale
            sh_ref[...] = b_ref[pl.ds(p - 1, 1), :] - mean * scale

        s1_ref[...] = jnp.zeros_like(s1_ref)
        s2_ref[...] = jnp.zeros_like(s2_ref)

    @pl.when(p == 0)
    def _stream_in():
        blk = x_ref[...]
        h_ref[pl.ds(c * chunk, chunk), :] = blk
        s1_ref[...] += jnp.sum(blk, axis=0, keepdims=True)
        s2_ref[...] += jnp.sum(blk * blk, axis=0, keepdims=True)

    @pl.when((p > 0) & (p < n_phases - 1))
    def _middle():
        blk = h_ref[pl.ds(c * chunk, chunk), :]
        blk = jnp.maximum(blk * sc_ref[...] + sh_ref[...], 0.0)
        h_ref[pl.ds(c * chunk, chunk), :] = blk
        s1_ref[...] += jnp.sum(blk, axis=0, keepdims=True)
        s2_ref[...] += jnp.sum(blk * blk, axis=0, keepdims=True)

    @pl.when(p == n_phases - 1)
    def _stream_out():
        blk = h_ref[pl.ds(c * chunk, chunk), :]
        o_ref[...] = jnp.maximum(blk * sc_ref[...] + sh_ref[...], 0.0)


def kernel(x, edge_index, instr_vectors, batch, Ws, bs, gammas, betas):
    del edge_index, instr_vectors, batch, Ws, bs
    n_rows, d = x.shape
    n_bn = gammas.shape[0]
    chunk = n_rows // _N_CHUNKS
    n_phases = n_bn + 1
    nc = _N_CHUNKS
    return pl.pallas_call(
        _bn_relu_chain_kernel,
        grid=(n_phases, nc),
        in_specs=[
            pl.BlockSpec((chunk, d),
                         lambda p, c: (jnp.where(p == 0, c, nc - 1), 0)),
            pl.BlockSpec((n_bn, d), lambda p, c: (0, 0)),
            pl.BlockSpec((n_bn, d), lambda p, c: (0, 0)),
        ],
        out_specs=pl.BlockSpec(
            (chunk, d), lambda p, c: (jnp.where(p == n_phases - 1, c, 0), 0)),
        scratch_shapes=[
            pltpu.VMEM((n_rows, d), jnp.float32),
            pltpu.VMEM((1, d), jnp.float32),
            pltpu.VMEM((1, d), jnp.float32),
            pltpu.VMEM((1, d), jnp.float32),
            pltpu.VMEM((1, d), jnp.float32),
        ],
        out_shape=jax.ShapeDtypeStruct(x.shape, x.dtype),
        compiler_params=pltpu.CompilerParams(
            dimension_semantics=("arbitrary", "arbitrary")),
    )(x, gammas, betas)


# speedup vs baseline: 1.4439x; 1.4439x over previous
import jax
import jax.numpy as jnp
from jax.experimental import pallas as pl
from jax.experimental.pallas import tpu as pltpu

_EPS = 1e-5
_NC = 5


def _bn_relu_chain_kernel(x_ref, g_ref, b_ref, o_ref,
                          h_ref, s1_ref, s2_ref, sc_ref, sh_ref):
    s = pl.program_id(0)
    n_rows = jnp.float32(h_ref.shape[0])
    chunk = x_ref.shape[0]
    n_bn = g_ref.shape[0]
    first_mid = _NC
    first_out = _NC + n_bn - 1

    @pl.when((s >= first_mid) & (s <= first_out))
    def _finalize_stats():
        layer = s - first_mid
        mean = s1_ref[...] / n_rows
        var = s2_ref[...] / n_rows - mean * mean
        scale = jax.lax.rsqrt(var + _EPS) * g_ref[pl.ds(layer, 1), :]
        sc_ref[...] = scale
        sh_ref[...] = b_ref[pl.ds(layer, 1), :] - mean * scale
        s1_ref[...] = jnp.zeros_like(s1_ref)
        s2_ref[...] = jnp.zeros_like(s2_ref)

    @pl.when(s < first_mid)
    def _stream_in():
        @pl.when(s == 0)
        def _init():
            s1_ref[...] = jnp.zeros_like(s1_ref)
            s2_ref[...] = jnp.zeros_like(s2_ref)
        blk = x_ref[...]
        h_ref[pl.ds(s * chunk, chunk), :] = blk
        s1_ref[...] += jnp.sum(blk, axis=0, keepdims=True)
        s2_ref[...] += jnp.sum(blk * blk, axis=0, keepdims=True)

    @pl.when((s >= first_mid) & (s < first_out))
    def _middle():
        blk = h_ref[...]
        blk = jnp.maximum(blk * sc_ref[...] + sh_ref[...], 0.0)
        h_ref[...] = blk
        s1_ref[...] = jnp.sum(blk, axis=0, keepdims=True)
        s2_ref[...] = jnp.sum(blk * blk, axis=0, keepdims=True)

    @pl.when(s >= first_out)
    def _stream_out():
        c = s - first_out
        blk = h_ref[pl.ds(c * chunk, chunk), :]
        o_ref[...] = jnp.maximum(blk * sc_ref[...] + sh_ref[...], 0.0)


def kernel(x, edge_index, instr_vectors, batch, Ws, bs, gammas, betas):
    del edge_index, instr_vectors, batch, Ws, bs
    n_rows, d = x.shape
    n_bn = gammas.shape[0]
    chunk = n_rows // _NC
    n_steps = 2 * _NC + n_bn - 1
    first_out = _NC + n_bn - 1
    return pl.pallas_call(
        _bn_relu_chain_kernel,
        grid=(n_steps,),
        in_specs=[
            pl.BlockSpec((chunk, d),
                         lambda s: (jnp.minimum(s, _NC - 1), 0)),
            pl.BlockSpec((n_bn, d), lambda s: (0, 0)),
            pl.BlockSpec((n_bn, d), lambda s: (0, 0)),
        ],
        out_specs=pl.BlockSpec(
            (chunk, d),
            lambda s: (jnp.maximum(s - first_out, 0), 0)),
        scratch_shapes=[
            pltpu.VMEM((n_rows, d), jnp.float32),
            pltpu.VMEM((1, d), jnp.float32),
            pltpu.VMEM((1, d), jnp.float32),
            pltpu.VMEM((1, d), jnp.float32),
            pltpu.VMEM((1, d), jnp.float32),
        ],
        out_shape=jax.ShapeDtypeStruct(x.shape, x.dtype),
        compiler_params=pltpu.CompilerParams(
            dimension_semantics=("arbitrary",)),
    )(x, gammas, betas)


# speedup vs baseline: 1.6215x; 1.1230x over previous
import jax
import jax.numpy as jnp
from jax.experimental import pallas as pl
from jax.experimental.pallas import tpu as pltpu

_EPS = 1e-5
_NC = 5


def _bn_relu_chain_kernel(x_hbm, g_ref, b_ref, o_hbm,
                          h_ref, in_sems, out_sems):
    n_rows = h_ref.shape[0]
    n = jnp.float32(n_rows)
    n_bn = g_ref.shape[0]
    ch = n_rows // _NC

    in_cps = [
        pltpu.make_async_copy(x_hbm.at[pl.ds(c * ch, ch), :],
                              h_ref.at[pl.ds(c * ch, ch), :],
                              in_sems.at[c])
        for c in range(_NC)
    ]
    for cp in in_cps:
        cp.start()

    s1 = jnp.zeros((1, h_ref.shape[1]), jnp.float32)
    s2 = jnp.zeros((1, h_ref.shape[1]), jnp.float32)
    for c in range(_NC):
        in_cps[c].wait()
        blk = h_ref[c * ch:(c + 1) * ch, :]
        s1 = s1 + jnp.sum(blk, axis=0, keepdims=True)
        s2 = s2 + jnp.sum(blk * blk, axis=0, keepdims=True)

    out_cps = []
    for layer in range(n_bn):
        mean = s1 / n
        var = s2 / n - mean * mean
        scale = jax.lax.rsqrt(var + _EPS) * g_ref[layer:layer + 1, :]
        shift = b_ref[layer:layer + 1, :] - mean * scale
        if layer < n_bn - 1:
            h = jnp.maximum(h_ref[...] * scale + shift, 0.0)
            h_ref[...] = h
            s1 = jnp.sum(h, axis=0, keepdims=True)
            s2 = jnp.sum(h * h, axis=0, keepdims=True)
        else:
            for c in range(_NC):
                sl = pl.ds(c * ch, ch)
                h_ref[sl, :] = jnp.maximum(
                    h_ref[sl, :] * scale + shift, 0.0)
                cp = pltpu.make_async_copy(h_ref.at[sl, :],
                                           o_hbm.at[sl, :],
                                           out_sems.at[c])
                cp.start()
                out_cps.append(cp)
    for cp in out_cps:
        cp.wait()


def kernel(x, edge_index, instr_vectors, batch, Ws, bs, gammas, betas):
    del edge_index, instr_vectors, batch, Ws, bs
    n_rows, d = x.shape
    n_bn = gammas.shape[0]
    return pl.pallas_call(
        _bn_relu_chain_kernel,
        in_specs=[
            pl.BlockSpec(memory_space=pltpu.MemorySpace.HBM),
            pl.BlockSpec(memory_space=pltpu.MemorySpace.VMEM),
            pl.BlockSpec(memory_space=pltpu.MemorySpace.VMEM),
        ],
        out_specs=pl.BlockSpec(memory_space=pltpu.MemorySpace.HBM),
        scratch_shapes=[
            pltpu.VMEM((n_rows, d), jnp.float32),
            pltpu.SemaphoreType.DMA((_NC,)),
            pltpu.SemaphoreType.DMA((_NC,)),
        ],
        out_shape=jax.ShapeDtypeStruct(x.shape, x.dtype),
    )(x, gammas, betas)
